# baseline (speedup 1.0000x reference)
import functools

import jax
import jax.numpy as jnp
from jax import lax
from jax.experimental import pallas as pl
from jax.experimental.pallas import tpu as pltpu
from jax.experimental.pallas import tpu_sc as plsc

D = 64
NC = 2
NS = 16
NW = NC * NS

BBLK = 128
LPU = 2
ROWS = BBLK * LPU


@functools.partial(jax.jit, static_argnames=("B", "L"))
def _sc_embed(idx, table, B, L):
    cb_per_w = (B // BBLK) // NW
    units_l = L // LPU
    units_per_w = cb_per_w * units_l
    assert units_per_w % 2 == 0
    mesh = plsc.VectorSubcoreMesh(core_axis_name="c", subcore_axis_name="s")

    @functools.partial(
        pl.kernel,
        mesh=mesh,
        compiler_params=pltpu.CompilerParams(
            use_tc_tiling_on_sc=False, needs_layout_passes=False
        ),
        out_type=jax.ShapeDtypeStruct((L, 8, B // BBLK, 8, BBLK), jnp.float32),
        scratch_types=[
            pltpu.VMEM((BBLK, L), jnp.int32),
            pltpu.VMEM((ROWS,), jnp.int32),
            pltpu.VMEM((ROWS,), jnp.int32),
            pltpu.VMEM((ROWS, D), jnp.float32),
            pltpu.VMEM((ROWS, D), jnp.float32),
            pltpu.VMEM((LPU, 8, 8, BBLK), jnp.float32),
            pltpu.VMEM((LPU, 8, 8, BBLK), jnp.float32),
            pltpu.SemaphoreType.DMA,
            pltpu.SemaphoreType.DMA,
            pltpu.SemaphoreType.DMA,
            pltpu.SemaphoreType.DMA,
        ],
    )
    def k(idx_hbm, table_hbm, out_hbm, iu, gi0, gi1, gb0, gb1, tb0, tb1,
          gs0, gs1, ws0, ws1):
        gi = (gi0, gi1)
        gb = (gb0, gb1)
        tb = (tb0, tb1)
        gsem = (gs0, gs1)
        wsem = (ws0, ws1)
        wid = lax.axis_index("s") * NC + lax.axis_index("c")
        iota = lax.iota(jnp.int32, 16)

        def coords(u):
            cb = u // units_l
            lb2 = u - cb * units_l
            b0 = pl.multiple_of((wid * cb_per_w + cb) * BBLK, 8)
            c = wid * cb_per_w + cb
            l = lb2 * LPU
            return lb2, b0, c, l

        def stage_idx(u):
            lb2, b0, _, _ = coords(u)

            @pl.when(lb2 == 0)
            def _():
                pltpu.sync_copy(idx_hbm.at[pl.ds(b0, BBLK), :], iu.at[...])

        def build_gi(u, b):
            _, _, _, l = coords(u)
            for lr in range(LPU):
                lsp = jnp.zeros((16,), jnp.int32) + (l + lr)
                for kk in range(BBLK // 16):
                    bvec = iota + (kk * 16)
                    v = plsc.load_gather(iu, [bvec, lsp])
                    gi[b][pl.ds(lr * BBLK + kk * 16, 16)] = v + 1

        def fire_gathers(b):
            for lr in range(LPU):
                pltpu.async_copy(
                    table_hbm.at[gi[b].at[pl.ds(lr * BBLK, BBLK)]],
                    gb[b].at[pl.ds(lr * BBLK, BBLK), :],
                    gsem[b],
                )

        def drain_gathers(b):
            for lr in range(LPU):
                pltpu.make_async_copy(
                    table_hbm.at[gi[b].at[pl.ds(lr * BBLK, BBLK)]],
                    gb[b].at[pl.ds(lr * BBLK, BBLK), :],
                    gsem[b],
                ).wait()

        def transpose(b):
            @pl.loop(0, D)
            def _d(d):
                t = d // 8
                dr = d - t * 8
                dsp = jnp.zeros((16,), jnp.int32) + d
                for lr in range(LPU):
                    for kk in range(BBLK // 16):
                        rvec = iota + (lr * BBLK + kk * 16)
                        v = plsc.load_gather(gb[b], [rvec, dsp])
                        tb[b][lr, t, dr, pl.ds(kk * 16, 16)] = v

        def fire_out(u, b):
            _, _, c, l = coords(u)
            for lr in range(LPU):
                pltpu.async_copy(
                    tb[b].at[lr], out_hbm.at[l + lr, :, c, :, :], wsem[b]
                )

        def wait_out(u, b):
            _, _, c, l = coords(u)
            for lr in range(LPU):
                pltpu.make_async_copy(
                    tb[b].at[lr], out_hbm.at[l + lr, :, c, :, :], wsem[b]
                ).wait()

        stage_idx(0)
        build_gi(0, 0)
        fire_gathers(0)

        @pl.loop(0, units_per_w, step=2)
        def _pair(u0):
            for bi in range(2):
                u = u0 + bi
                b = bi
                nb = 1 - bi

                @pl.when(u + 1 < units_per_w)
                def _prefetch():
                    stage_idx(u + 1)
                    build_gi(u + 1, nb)

                @pl.when(u >= 1)
                def _free():
                    wait_out(u - 1, nb)

                @pl.when(u + 1 < units_per_w)
                def _fire():
                    fire_gathers(nb)

                drain_gathers(b)
                transpose(b)
                fire_out(u, b)

        wait_out(units_per_w - 1, 1)

    return k(idx, table)


def kernel(channel_idx, table):
    B, L = channel_idx.shape
    a = _sc_embed(channel_idx.astype(jnp.int32), table, B, L)
    return a.transpose(2, 4, 0, 1, 3).reshape(B, L, 1, D)


# speedup vs baseline: 1.3060x; 1.3060x over previous
import functools

import jax
import jax.numpy as jnp
from jax import lax
from jax.experimental import pallas as pl
from jax.experimental.pallas import tpu as pltpu
from jax.experimental.pallas import tpu_sc as plsc

D = 64
NC = 2
NS = 16
NW = NC * NS

BBLK = 128
LPU = 2
ROWS = BBLK * LPU


@functools.partial(jax.jit, static_argnames=("B", "L"))
def _sc_embed(idx, table, B, L):
    cb_per_w = (B // BBLK) // NW
    units_l = L // LPU
    units_per_w = cb_per_w * units_l
    assert units_per_w % 2 == 0
    mesh = plsc.VectorSubcoreMesh(core_axis_name="c", subcore_axis_name="s")

    @functools.partial(
        pl.kernel,
        mesh=mesh,
        compiler_params=pltpu.CompilerParams(
            use_tc_tiling_on_sc=False, needs_layout_passes=False
        ),
        out_type=jax.ShapeDtypeStruct((L, 8, B // BBLK, 8, BBLK), jnp.float32),
        scratch_types=[
            pltpu.VMEM((BBLK, L), jnp.int32),
            pltpu.VMEM((ROWS,), jnp.int32),
            pltpu.VMEM((ROWS,), jnp.int32),
            pltpu.VMEM((ROWS, D), jnp.float32),
            pltpu.VMEM((ROWS, D), jnp.float32),
            pltpu.VMEM((LPU, 8, 8, BBLK), jnp.float32),
            pltpu.VMEM((LPU, 8, 8, BBLK), jnp.float32),
            pltpu.SemaphoreType.DMA,
            pltpu.SemaphoreType.DMA,
            pltpu.SemaphoreType.DMA,
            pltpu.SemaphoreType.DMA,
        ],
    )
    def k(idx_hbm, table_hbm, out_hbm, iu, gi0, gi1, gb0, gb1, tb0, tb1,
          gs0, gs1, ws0, ws1):
        gi = (gi0, gi1)
        gb = (gb0, gb1)
        tb = (tb0, tb1)
        gsem = (gs0, gs1)
        wsem = (ws0, ws1)
        wid = lax.axis_index("s") * NC + lax.axis_index("c")
        iota = lax.iota(jnp.int32, 16)

        def coords(u):
            cb = u // units_l
            lb2 = u - cb * units_l
            b0 = pl.multiple_of((wid * cb_per_w + cb) * BBLK, 8)
            c = wid * cb_per_w + cb
            l = lb2 * LPU
            return lb2, b0, c, l

        def stage_idx(u):
            lb2, b0, _, _ = coords(u)

            @pl.when(lb2 == 0)
            def _():
                pltpu.sync_copy(idx_hbm.at[pl.ds(b0, BBLK), :], iu.at[...])

        def build_gi(u, b):
            _, _, _, l = coords(u)
            vals = []
            for lr in range(LPU):
                lsp = jnp.zeros((16,), jnp.int32) + (l + lr)
                for kk in range(BBLK // 16):
                    bvec = iota + (kk * 16)
                    vals.append(plsc.load_gather(iu, [bvec, lsp]) + 1)
            i = 0
            for lr in range(LPU):
                for kk in range(BBLK // 16):
                    gi[b][pl.ds(lr * BBLK + kk * 16, 16)] = vals[i]
                    i += 1

        def fire_gathers(b):
            for lr in range(LPU):
                pltpu.async_copy(
                    table_hbm.at[gi[b].at[pl.ds(lr * BBLK, BBLK)]],
                    gb[b].at[pl.ds(lr * BBLK, BBLK), :],
                    gsem[b],
                )

        def drain_gathers(b):
            for lr in range(LPU):
                pltpu.make_async_copy(
                    table_hbm.at[gi[b].at[pl.ds(lr * BBLK, BBLK)]],
                    gb[b].at[pl.ds(lr * BBLK, BBLK), :],
                    gsem[b],
                ).wait()

        def transpose(b):
            @plsc.parallel_loop(0, D, unroll=4)
            def _d(d):
                t = d // 8
                dr = d - t * 8
                dsp = jnp.zeros((16,), jnp.int32) + d
                vals = []
                for lr in range(LPU):
                    for kk in range(BBLK // 16):
                        rvec = iota + (lr * BBLK + kk * 16)
                        vals.append(plsc.load_gather(gb[b], [rvec, dsp]))
                i = 0
                for lr in range(LPU):
                    for kk in range(BBLK // 16):
                        tb[b][lr, t, dr, pl.ds(kk * 16, 16)] = vals[i]
                        i += 1

        def fire_out(u, b):
            _, _, c, l = coords(u)
            for lr in range(LPU):
                pltpu.async_copy(
                    tb[b].at[lr], out_hbm.at[l + lr, :, c, :, :], wsem[b]
                )

        def wait_out(u, b):
            _, _, c, l = coords(u)
            for lr in range(LPU):
                pltpu.make_async_copy(
                    tb[b].at[lr], out_hbm.at[l + lr, :, c, :, :], wsem[b]
                ).wait()

        stage_idx(0)
        build_gi(0, 0)
        fire_gathers(0)

        @pl.loop(0, units_per_w, step=2)
        def _pair(u0):
            for bi in range(2):
                u = u0 + bi
                b = bi
                nb = 1 - bi

                @pl.when(u + 1 < units_per_w)
                def _prefetch():
                    stage_idx(u + 1)
                    build_gi(u + 1, nb)

                @pl.when(u >= 1)
                def _free():
                    wait_out(u - 1, nb)

                @pl.when(u + 1 < units_per_w)
                def _fire():
                    fire_gathers(nb)

                drain_gathers(b)
                transpose(b)
                fire_out(u, b)

        wait_out(units_per_w - 1, 1)

    return k(idx, table)


def kernel(channel_idx, table):
    B, L = channel_idx.shape
    a = _sc_embed(channel_idx.astype(jnp.int32), table, B, L)
    return a.transpose(2, 4, 0, 1, 3).reshape(B, L, 1, D)


# speedup vs baseline: 7.2469x; 5.5490x over previous
import functools

import jax
import jax.numpy as jnp
from jax import lax
from jax.experimental import pallas as pl
from jax.experimental.pallas import tpu as pltpu
from jax.experimental.pallas import tpu_sc as plsc

D = 64
NC = 2
NS = 16
NW = NC * NS

BBLK = 128
LPU = 2
ROWS = BBLK * LPU


@functools.partial(jax.jit, static_argnames=("B", "L"))
def _sc_embed(idx, table, B, L):
    cb_per_w = (B // BBLK) // NW
    units_l = L // LPU
    units_per_w = cb_per_w * units_l
    assert units_per_w % 2 == 0
    mesh = plsc.VectorSubcoreMesh(core_axis_name="c", subcore_axis_name="s")

    @functools.partial(
        pl.kernel,
        mesh=mesh,
        compiler_params=pltpu.CompilerParams(
            use_tc_tiling_on_sc=False, needs_layout_passes=False
        ),
        out_type=jax.ShapeDtypeStruct((L, 8, B // BBLK, 8, BBLK), jnp.float32),
        scratch_types=[
            pltpu.VMEM((BBLK, L + 1), jnp.int32),
            pltpu.VMEM((ROWS,), jnp.int32),
            pltpu.VMEM((ROWS,), jnp.int32),
            pltpu.VMEM((ROWS, D), jnp.float32),
            pltpu.VMEM((ROWS, D), jnp.float32),
            pltpu.VMEM((LPU, 8, 8, BBLK + 1), jnp.float32),
            pltpu.VMEM((LPU, 8, 8, BBLK + 1), jnp.float32),
            pltpu.SemaphoreType.DMA,
            pltpu.SemaphoreType.DMA,
            pltpu.SemaphoreType.DMA,
            pltpu.SemaphoreType.DMA,
        ],
    )
    def k(idx_hbm, table_hbm, out_hbm, iu, gi0, gi1, gb0, gb1, tb0, tb1,
          gs0, gs1, ws0, ws1):
        gi = (gi0, gi1)
        gb = (gb0, gb1)
        tb = (tb0, tb1)
        gsem = (gs0, gs1)
        wsem = (ws0, ws1)
        wid = lax.axis_index("s") * NC + lax.axis_index("c")
        iota = lax.iota(jnp.int32, 16)

        def coords(u):
            cb = u // units_l
            lb2 = u - cb * units_l
            b0 = pl.multiple_of((wid * cb_per_w + cb) * BBLK, 8)
            c = wid * cb_per_w + cb
            l = lb2 * LPU
            return lb2, b0, c, l

        def stage_idx(u):
            lb2, b0, _, _ = coords(u)

            @pl.when(lb2 == 0)
            def _():
                pltpu.sync_copy(
                    idx_hbm.at[pl.ds(b0, BBLK), :], iu.at[:, pl.ds(0, L)]
                )

        def build_gi(u, b):
            _, _, _, l = coords(u)
            vals = []
            for lr in range(LPU):
                lsp = jnp.zeros((16,), jnp.int32) + (l + lr)
                for kk in range(BBLK // 16):
                    bvec = iota + (kk * 16)
                    vals.append(plsc.load_gather(iu, [bvec, lsp]) + 1)
            i = 0
            for lr in range(LPU):
                for kk in range(BBLK // 16):
                    gi[b][pl.ds(lr * BBLK + kk * 16, 16)] = vals[i]
                    i += 1

        def fire_gathers(b):
            for lr in range(LPU):
                pltpu.async_copy(
                    table_hbm.at[gi[b].at[pl.ds(lr * BBLK, BBLK)]],
                    gb[b].at[pl.ds(lr * BBLK, BBLK), :],
                    gsem[b],
                )

        def drain_gathers(b):
            for lr in range(LPU):
                pltpu.make_async_copy(
                    table_hbm.at[gi[b].at[pl.ds(lr * BBLK, BBLK)]],
                    gb[b].at[pl.ds(lr * BBLK, BBLK), :],
                    gsem[b],
                ).wait()

        def transpose(b):
            for lr in range(LPU):
                @plsc.parallel_loop(0, BBLK, unroll=2)
                def _r(r):
                    bsp = jnp.zeros((16,), jnp.int32) + r
                    for d0 in range(0, D, 16):
                        v = gb[b][lr * BBLK + r, pl.ds(d0, 16)]
                        tvec = (d0 + iota) // 8
                        drvec = (d0 + iota) % 8
                        plsc.store_scatter(tb[b].at[lr], [tvec, drvec, bsp], v)

        def fire_out(u, b):
            _, _, c, l = coords(u)
            for lr in range(LPU):
                pltpu.async_copy(
                    tb[b].at[lr, :, :, pl.ds(0, BBLK)],
                    out_hbm.at[l + lr, :, c, :, :],
                    wsem[b],
                )

        def wait_out(u, b):
            _, _, c, l = coords(u)
            for lr in range(LPU):
                pltpu.make_async_copy(
                    tb[b].at[lr, :, :, pl.ds(0, BBLK)],
                    out_hbm.at[l + lr, :, c, :, :],
                    wsem[b],
                ).wait()

        stage_idx(0)
        build_gi(0, 0)
        fire_gathers(0)

        @pl.loop(0, units_per_w, step=2)
        def _pair(u0):
            for bi in range(2):
                u = u0 + bi
                b = bi
                nb = 1 - bi

                @pl.when(u + 1 < units_per_w)
                def _prefetch():
                    stage_idx(u + 1)
                    build_gi(u + 1, nb)

                @pl.when(u >= 1)
                def _free():
                    wait_out(u - 1, nb)

                @pl.when(u + 1 < units_per_w)
                def _fire():
                    fire_gathers(nb)

                drain_gathers(b)
                transpose(b)
                fire_out(u, b)

        wait_out(units_per_w - 1, 1)

    return k(idx, table)


def kernel(channel_idx, table):
    B, L = channel_idx.shape
    a = _sc_embed(channel_idx.astype(jnp.int32), table, B, L)
    return a.transpose(2, 4, 0, 1, 3).reshape(B, L, 1, D)


# speedup vs baseline: 7.5593x; 1.0431x over previous
import functools

import jax
import jax.numpy as jnp
from jax import lax
from jax.experimental import pallas as pl
from jax.experimental.pallas import tpu as pltpu
from jax.experimental.pallas import tpu_sc as plsc

D = 64
NC = 2
NS = 16
NW = NC * NS

BBLK = 128
LPU = 2
ROWS = BBLK * LPU


@functools.partial(jax.jit, static_argnames=("B", "L"))
def _sc_embed(idx, table, B, L):
    cb_per_w = (B // BBLK) // NW
    units_l = L // LPU
    units_per_w = cb_per_w * units_l
    assert units_per_w % 2 == 0
    mesh = plsc.VectorSubcoreMesh(core_axis_name="c", subcore_axis_name="s")

    @functools.partial(
        pl.kernel,
        mesh=mesh,
        compiler_params=pltpu.CompilerParams(
            use_tc_tiling_on_sc=False, needs_layout_passes=False
        ),
        out_type=jax.ShapeDtypeStruct((L, 8, B // BBLK, 8, BBLK), jnp.float32),
        scratch_types=[
            pltpu.VMEM((L // 8, 8, BBLK), jnp.int32),
            pltpu.VMEM((ROWS,), jnp.int32),
            pltpu.VMEM((ROWS,), jnp.int32),
            pltpu.VMEM((ROWS, D), jnp.float32),
            pltpu.VMEM((ROWS, D), jnp.float32),
            pltpu.VMEM((LPU, 8, 8, BBLK + 1), jnp.float32),
            pltpu.VMEM((LPU, 8, 8, BBLK + 1), jnp.float32),
            pltpu.SemaphoreType.DMA,
            pltpu.SemaphoreType.DMA,
            pltpu.SemaphoreType.DMA,
            pltpu.SemaphoreType.DMA,
        ],
    )
    def k(idx_hbm, table_hbm, out_hbm, iu, gi0, gi1, gb0, gb1, tb0, tb1,
          gs0, gs1, ws0, ws1):
        gi = (gi0, gi1)
        gb = (gb0, gb1)
        tb = (tb0, tb1)
        gsem = (gs0, gs1)
        wsem = (ws0, ws1)
        wid = lax.axis_index("s") * NC + lax.axis_index("c")
        iota = lax.iota(jnp.int32, 16)

        def coords(u):
            cb = u // units_l
            lb2 = u - cb * units_l
            b0 = pl.multiple_of((wid * cb_per_w + cb) * BBLK, 8)
            c = wid * cb_per_w + cb
            l = lb2 * LPU
            return lb2, b0, c, l

        def stage_idx(u):
            lb2, _, c, _ = coords(u)

            @pl.when(lb2 == 0)
            def _():
                pltpu.sync_copy(idx_hbm.at[:, c, :, :], iu.at[...])

        def build_gi(u, b):
            _, _, _, l = coords(u)
            vals = []
            for lr in range(LPU):
                lq = (l + lr) // 8
                l8 = (l + lr) - lq * 8
                for kk in range(BBLK // 16):
                    vals.append(iu[lq, l8, pl.ds(kk * 16, 16)] + 1)
            i = 0
            for lr in range(LPU):
                for kk in range(BBLK // 16):
                    gi[b][pl.ds(lr * BBLK + kk * 16, 16)] = vals[i]
                    i += 1

        def fire_gathers(b):
            for lr in range(LPU):
                pltpu.async_copy(
                    table_hbm.at[gi[b].at[pl.ds(lr * BBLK, BBLK)]],
                    gb[b].at[pl.ds(lr * BBLK, BBLK), :],
                    gsem[b],
                )

        def drain_gathers(b):
            for lr in range(LPU):
                pltpu.make_async_copy(
                    table_hbm.at[gi[b].at[pl.ds(lr * BBLK, BBLK)]],
                    gb[b].at[pl.ds(lr * BBLK, BBLK), :],
                    gsem[b],
                ).wait()

        def transpose(b):
            for lr in range(LPU):
                @plsc.parallel_loop(0, BBLK, unroll=2)
                def _r(r):
                    bsp = jnp.zeros((16,), jnp.int32) + r
                    for d0 in range(0, D, 16):
                        v = gb[b][lr * BBLK + r, pl.ds(d0, 16)]
                        tvec = (d0 + iota) // 8
                        drvec = (d0 + iota) % 8
                        plsc.store_scatter(tb[b].at[lr], [tvec, drvec, bsp], v)

        def fire_out(u, b):
            _, _, c, l = coords(u)
            for lr in range(LPU):
                pltpu.async_copy(
                    tb[b].at[lr, :, :, pl.ds(0, BBLK)],
                    out_hbm.at[l + lr, :, c, :, :],
                    wsem[b],
                )

        def wait_out(u, b):
            _, _, c, l = coords(u)
            for lr in range(LPU):
                pltpu.make_async_copy(
                    tb[b].at[lr, :, :, pl.ds(0, BBLK)],
                    out_hbm.at[l + lr, :, c, :, :],
                    wsem[b],
                ).wait()

        stage_idx(0)
        build_gi(0, 0)
        fire_gathers(0)

        @pl.loop(0, units_per_w, step=2)
        def _pair(u0):
            for bi in range(2):
                u = u0 + bi
                b = bi
                nb = 1 - bi

                @pl.when(u + 1 < units_per_w)
                def _prefetch():
                    stage_idx(u + 1)
                    build_gi(u + 1, nb)

                @pl.when(u >= 1)
                def _free():
                    wait_out(u - 1, nb)

                @pl.when(u + 1 < units_per_w)
                def _fire():
                    fire_gathers(nb)

                drain_gathers(b)
                transpose(b)
                fire_out(u, b)

        wait_out(units_per_w - 1, 1)

    return k(idx, table)


def kernel(channel_idx, table):
    B, L = channel_idx.shape
    idx_t = (
        channel_idx.astype(jnp.int32)
        .T.reshape(L // 8, 8, B // BBLK, BBLK)
        .transpose(0, 2, 1, 3)
    )
    a = _sc_embed(idx_t, table, B, L)
    return a.transpose(2, 4, 0, 1, 3).reshape(B, L, 1, D)


# speedup vs baseline: 7.5801x; 1.0027x over previous
import functools

import jax
import jax.numpy as jnp
from jax import lax
from jax.experimental import pallas as pl
from jax.experimental.pallas import tpu as pltpu
from jax.experimental.pallas import tpu_sc as plsc

D = 64
NC = 2
NS = 16
NW = NC * NS

BBLK = 128
LPU = 2
ROWS = BBLK * LPU


@functools.partial(jax.jit, static_argnames=("B", "L"))
def _sc_embed(idx, table, B, L):
    cb_per_w = (B // BBLK) // NW
    units_l = L // LPU
    units_per_w = cb_per_w * units_l
    assert units_per_w % 2 == 0
    mesh = plsc.VectorSubcoreMesh(core_axis_name="c", subcore_axis_name="s")

    @functools.partial(
        pl.kernel,
        mesh=mesh,
        compiler_params=pltpu.CompilerParams(
            use_tc_tiling_on_sc=False, needs_layout_passes=False
        ),
        out_type=jax.ShapeDtypeStruct((L, 8, B // BBLK, 8, BBLK), jnp.float32),
        scratch_types=[
            pltpu.VMEM((L // 8, 8, BBLK), jnp.int32),
            pltpu.VMEM((ROWS,), jnp.int32),
            pltpu.VMEM((ROWS,), jnp.int32),
            pltpu.VMEM((ROWS, D), jnp.float32),
            pltpu.VMEM((ROWS, D), jnp.float32),
            pltpu.VMEM((LPU, 8, 8, BBLK + 1), jnp.float32),
            pltpu.VMEM((LPU, 8, 8, BBLK + 1), jnp.float32),
            pltpu.SemaphoreType.DMA,
            pltpu.SemaphoreType.DMA,
            pltpu.SemaphoreType.DMA,
            pltpu.SemaphoreType.DMA,
        ],
    )
    def k(idx_hbm, table_hbm, out_hbm, iu, gi0, gi1, gb0, gb1, tb0, tb1,
          gs0, gs1, ws0, ws1):
        gi = (gi0, gi1)
        gb = (gb0, gb1)
        tb = (tb0, tb1)
        gsem = (gs0, gs1)
        wsem = (ws0, ws1)
        wid = lax.axis_index("s") * NC + lax.axis_index("c")
        iota = lax.iota(jnp.int32, 16)

        def coords(u):
            cb = u // units_l
            lb2 = u - cb * units_l
            b0 = pl.multiple_of((wid * cb_per_w + cb) * BBLK, 8)
            c = wid * cb_per_w + cb
            l = lb2 * LPU
            return lb2, b0, c, l

        def stage_idx(u):
            lb2, _, c, _ = coords(u)

            @pl.when(lb2 == 0)
            def _():
                pltpu.sync_copy(idx_hbm.at[:, c, :, :], iu.at[...])

        def build_gi(u, b):
            _, _, _, l = coords(u)
            vals = []
            for lr in range(LPU):
                lq = (l + lr) // 8
                l8 = (l + lr) - lq * 8
                for kk in range(BBLK // 16):
                    vals.append(iu[lq, l8, pl.ds(kk * 16, 16)] + 1)
            i = 0
            for lr in range(LPU):
                for kk in range(BBLK // 16):
                    gi[b][pl.ds(lr * BBLK + kk * 16, 16)] = vals[i]
                    i += 1

        def fire_gathers(b):
            for lr in range(LPU):
                pltpu.async_copy(
                    table_hbm.at[gi[b].at[pl.ds(lr * BBLK, BBLK)]],
                    gb[b].at[pl.ds(lr * BBLK, BBLK), :],
                    gsem[b],
                )

        def drain_gathers(b):
            for lr in range(LPU):
                pltpu.make_async_copy(
                    table_hbm.at[gi[b].at[pl.ds(lr * BBLK, BBLK)]],
                    gb[b].at[pl.ds(lr * BBLK, BBLK), :],
                    gsem[b],
                ).wait()

        def transpose(b):
            for lr in range(LPU):
                @plsc.parallel_loop(0, BBLK, unroll=4)
                def _r(r):
                    bsp = jnp.zeros((16,), jnp.int32) + r
                    for d0 in range(0, D, 16):
                        v = gb[b][lr * BBLK + r, pl.ds(d0, 16)]
                        tvec = (d0 + iota) // 8
                        drvec = (d0 + iota) % 8
                        plsc.store_scatter(tb[b].at[lr], [tvec, drvec, bsp], v)

        def fire_out(u, b):
            _, _, c, l = coords(u)
            pltpu.async_copy(
                tb[b].at[:, :, :, pl.ds(0, BBLK)],
                out_hbm.at[pl.ds(l, LPU), :, c, :, :],
                wsem[b],
            )

        def wait_out(u, b):
            _, _, c, l = coords(u)
            pltpu.make_async_copy(
                tb[b].at[:, :, :, pl.ds(0, BBLK)],
                out_hbm.at[pl.ds(l, LPU), :, c, :, :],
                wsem[b],
            ).wait()

        stage_idx(0)
        build_gi(0, 0)
        fire_gathers(0)

        @pl.loop(0, units_per_w, step=2)
        def _pair(u0):
            for bi in range(2):
                u = u0 + bi
                b = bi
                nb = 1 - bi

                @pl.when(u + 1 < units_per_w)
                def _prefetch():
                    stage_idx(u + 1)
                    build_gi(u + 1, nb)

                @pl.when(u >= 1)
                def _free():
                    wait_out(u - 1, nb)

                @pl.when(u + 1 < units_per_w)
                def _fire():
                    fire_gathers(nb)

                drain_gathers(b)
                transpose(b)
                fire_out(u, b)

        wait_out(units_per_w - 1, 1)

    return k(idx, table)


def kernel(channel_idx, table):
    B, L = channel_idx.shape
    idx_t = (
        channel_idx.astype(jnp.int32)
        .T.reshape(L // 8, 8, B // BBLK, BBLK)
        .transpose(0, 2, 1, 3)
    )
    a = _sc_embed(idx_t, table, B, L)
    return a.transpose(2, 4, 0, 1, 3).reshape(B, L, 1, D)


# speedup vs baseline: 7.5857x; 1.0007x over previous
import functools

import jax
import jax.numpy as jnp
from jax import lax
from jax.experimental import pallas as pl
from jax.experimental.pallas import tpu as pltpu
from jax.experimental.pallas import tpu_sc as plsc

D = 64
NC = 2
NS = 16
NW = NC * NS

BBLK = 128
LPU = 2
ROWS = BBLK * LPU


@functools.partial(jax.jit, static_argnames=("B", "L"))
def _sc_embed(idx, table, B, L):
    cb_per_w = (B // BBLK) // NW
    units_l = L // LPU
    units_per_w = cb_per_w * units_l
    assert units_per_w % 2 == 0
    mesh = plsc.VectorSubcoreMesh(core_axis_name="c", subcore_axis_name="s")

    @functools.partial(
        pl.kernel,
        mesh=mesh,
        compiler_params=pltpu.CompilerParams(
            use_tc_tiling_on_sc=False, needs_layout_passes=False
        ),
        out_type=jax.ShapeDtypeStruct((L, 8, B // BBLK, 8, BBLK), jnp.float32),
        scratch_types=[
            pltpu.VMEM((L // 8, 8, BBLK), jnp.int32),
            pltpu.VMEM((ROWS,), jnp.int32),
            pltpu.VMEM((ROWS,), jnp.int32),
            pltpu.VMEM((ROWS, D), jnp.float32),
            pltpu.VMEM((ROWS, D), jnp.float32),
            pltpu.VMEM((LPU, 8, 8, BBLK + 1), jnp.float32),
            pltpu.VMEM((LPU, 8, 8, BBLK + 1), jnp.float32),
            pltpu.SemaphoreType.DMA,
            pltpu.SemaphoreType.DMA,
            pltpu.SemaphoreType.DMA,
            pltpu.SemaphoreType.DMA,
        ],
    )
    def k(idx_hbm, table_hbm, out_hbm, iu, gi0, gi1, gb0, gb1, tb0, tb1,
          gs0, gs1, ws0, ws1):
        gi = (gi0, gi1)
        gb = (gb0, gb1)
        tb = (tb0, tb1)
        gsem = (gs0, gs1)
        wsem = (ws0, ws1)
        wid = lax.axis_index("s") * NC + lax.axis_index("c")
        iota = lax.iota(jnp.int32, 16)

        def coords(u):
            cb = u // units_l
            lb2 = u - cb * units_l
            c = wid * cb_per_w + cb
            l = lb2 * LPU
            return lb2, c, l

        def stage_idx(u):
            lb2, c, _ = coords(u)

            @pl.when(lb2 == 0)
            def _():
                pltpu.sync_copy(idx_hbm.at[:, c, :, :], iu.at[...])

        def build_gi(u, b):
            _, _, l = coords(u)
            vals = []
            for lr in range(LPU):
                lq = (l + lr) // 8
                l8 = (l + lr) - lq * 8
                for kk in range(BBLK // 16):
                    vals.append(iu[lq, l8, pl.ds(kk * 16, 16)] + 1)
            i = 0
            for lr in range(LPU):
                for kk in range(BBLK // 16):
                    gi[b][pl.ds(lr * BBLK + kk * 16, 16)] = vals[i]
                    i += 1

        def fire_gathers(b):
            for lr in range(LPU):
                pltpu.async_copy(
                    table_hbm.at[gi[b].at[pl.ds(lr * BBLK, BBLK)]],
                    gb[b].at[pl.ds(lr * BBLK, BBLK), :],
                    gsem[b],
                )

        def drain_gathers(b):
            for lr in range(LPU):
                pltpu.make_async_copy(
                    table_hbm.at[gi[b].at[pl.ds(lr * BBLK, BBLK)]],
                    gb[b].at[pl.ds(lr * BBLK, BBLK), :],
                    gsem[b],
                ).wait()

        def transpose(b):
            for lr in range(LPU):
                @plsc.parallel_loop(0, BBLK, unroll=4)
                def _r(r):
                    bsp = jnp.zeros((16,), jnp.int32) + r
                    for d0 in range(0, D, 16):
                        v = gb[b][lr * BBLK + r, pl.ds(d0, 16)]
                        tvec = (d0 + iota) // 8
                        drvec = (d0 + iota) % 8
                        plsc.store_scatter(tb[b].at[lr], [tvec, drvec, bsp], v)

        def fire_out(u, b):
            _, c, l = coords(u)
            pltpu.async_copy(
                tb[b].at[:, :, :, pl.ds(0, BBLK)],
                out_hbm.at[pl.ds(l, LPU), :, c, :, :],
                wsem[b],
            )

        def wait_out(u, b):
            _, c, l = coords(u)
            pltpu.make_async_copy(
                tb[b].at[:, :, :, pl.ds(0, BBLK)],
                out_hbm.at[pl.ds(l, LPU), :, c, :, :],
                wsem[b],
            ).wait()

        stage_idx(0)
        build_gi(0, 0)
        fire_gathers(0)

        @pl.loop(0, units_per_w, step=2)
        def _pair(u0):
            for bi in range(2):
                u = u0 + bi
                b = bi
                nb = 1 - bi

                @pl.when(u + 1 < units_per_w)
                def _prefetch():
                    stage_idx(u + 1)
                    build_gi(u + 1, nb)

                @pl.when(u >= 1)
                def _free():
                    wait_out(u - 1, nb)

                @pl.when(u + 1 < units_per_w)
                def _fire():
                    fire_gathers(nb)

                drain_gathers(b)
                transpose(b)
                fire_out(u, b)

        wait_out(units_per_w - 1, 1)

    return k(idx, table)


def kernel(channel_idx, table):
    B, L = channel_idx.shape
    idx_t = (
        channel_idx.astype(jnp.int32)
        .T.reshape(L // 8, 8, B // BBLK, BBLK)
        .transpose(0, 2, 1, 3)
    )
    a = _sc_embed(idx_t, table, B, L)
    return a.transpose(2, 4, 0, 1, 3).reshape(B, L, 1, D)
